# Initial kernel scaffold; baseline (speedup 1.0000x reference)
import jax
import jax.numpy as jnp
from jax.experimental import pallas as pl


def kernel(x, seq_len):
    raise NotImplementedError("write your pallas kernel here")


# speedup vs baseline: 3.9198x; 3.9198x over previous
import functools

import jax
import jax.numpy as jnp
import numpy as np
from jax import lax
from jax.experimental import pallas as pl
from jax.experimental.pallas import tpu as pltpu
from jax.experimental.pallas import tpu_sc as plsc

B = 128
T = 32768
L = 16
NW = 32
ROWS_PER_W = B // NW
NSTEP = T // L
INT_MIN = np.int32(-2147483648)
M31 = np.int32(0x7FFFFFFF)


def _topk_mean_kernel(x_hbm, sl_hbm, out_hbm, vals_v, keys_v, hist_v,
                      tot_v, sl_v, res_v, sem):
    wid = lax.axis_index("s") * 2 + lax.axis_index("c")
    pltpu.sync_copy(sl_hbm, sl_v)

    lane = lax.iota(jnp.int32, 16)
    ones = jnp.ones((16,), jnp.int32)
    zf = jnp.zeros((16,), jnp.float32)
    zi = jnp.zeros((16,), jnp.int32)

    res = zf
    for j in range(ROWS_PER_W):
        r = wid * ROWS_PER_W + j
        pltpu.async_copy(x_hbm.at[r], vals_v, sem).wait()
        chunk = sl_v[pl.ds((r // 16) * 16, 16)]
        sl = jnp.sum(jnp.where(lane == (r % 16), chunk, 0))
        k = sl // 16 + 1

        def zero_body(i, _):
            hist_v[pl.ds(i * 16, 16)] = zi
            return 0
        lax.fori_loop(0, 256, zero_body, 0)

        def p0_body(i, _):
            v = vals_v[pl.ds(i * 16, 16)]
            m = plsc.bitcast(v, jnp.int32)
            m = jnp.where(m >= 0, m, m ^ M31)
            pos = i * 16 + lane
            m = jnp.where(pos < sl, m, INT_MIN)
            keys_v[pl.ds(i * 16, 16)] = m
            bkt = lax.shift_right_logical(m ^ INT_MIN, 24)
            plsc.addupdate_scatter(hist_v, [bkt * 16 + lane], ones)
            return 0
        lax.fori_loop(0, NSTEP, p0_body, 0)

        def reduce_hist(_):
            def red_body(c, _):
                bvec = (c * 16 + lane) * 16
                acc = zi
                for l in range(16):
                    acc = acc + plsc.load_gather(hist_v, [bvec + l])
                tot_v[pl.ds(c * 16, 16)] = acc
                return 0
            lax.fori_loop(0, 16, red_body, 0)

        def scan_hist(kr):
            def chunk_body(i, carry):
                above, bstar, krn = carry
                c = 15 - i
                tvec = tot_v[pl.ds(c * 16, 16)]
                cs = plsc.cumsum(tvec)
                ctot = cs[15]
                s = above + ctot - cs + tvec
                mask = s >= kr
                nhit = plsc.all_reduce_population_count(mask)[0]
                hit = (bstar < 0) & (nhit > 0)
                lane_star = nhit - 1
                cs_star = jnp.sum(jnp.where(lane == lane_star, cs, 0))
                cum_above = above + ctot - cs_star
                bstar = jnp.where(hit, c * 16 + lane_star, bstar)
                krn = jnp.where(hit, kr - cum_above, krn)
                return above + ctot, bstar, krn
            _, bstar, krn = lax.fori_loop(
                0, 16, chunk_body,
                (np.int32(0), np.int32(-1), np.int32(1)))
            return bstar, krn

        reduce_hist(None)
        bstar, kr = scan_hist(k)
        prefix = bstar << 24

        for lvl in range(1, 4):
            shift = 24 - 8 * lvl
            lax.fori_loop(0, 256, zero_body, 0)
            phi = lax.shift_right_logical(prefix, shift + 8)

            def lvl_body(i, _):
                m = keys_v[pl.ds(i * 16, 16)]
                uk = m ^ INT_MIN
                selm = lax.shift_right_logical(uk, shift + 8) == phi
                bkt = lax.shift_right_logical(uk, shift) & 255
                plsc.addupdate_scatter(hist_v, [bkt * 16 + lane], ones,
                                       mask=selm)
                return 0
            lax.fori_loop(0, NSTEP, lvl_body, 0)
            reduce_hist(None)
            bstar, kr = scan_hist(kr)
            prefix = prefix | (bstar << shift)

        tau_m = prefix ^ INT_MIN
        tau_b = jnp.where(tau_m >= 0, tau_m, tau_m ^ M31)
        tau_f = plsc.bitcast(jnp.broadcast_to(tau_b, (16,)), jnp.float32)

        def fin_body(i, acc):
            m = keys_v[pl.ds(i * 16, 16)]
            bb = jnp.where(m >= 0, m, m ^ M31)
            v = plsc.bitcast(bb, jnp.float32)
            return acc + jnp.where(m > tau_m, v, 0.0)
        acc = lax.fori_loop(0, NSTEP, fin_body, zf)

        sum_gt = jnp.sum(acc)
        krf = jnp.broadcast_to(kr, (16,)).astype(jnp.float32)
        kf = jnp.broadcast_to(k, (16,)).astype(jnp.float32)
        rj = (sum_gt + krf * tau_f) / kf
        res = jnp.where(lane == j, rj, res)

    res_v[...] = res
    pltpu.sync_copy(res_v, out_hbm.at[wid])


@functools.partial(
    pl.kernel,
    out_type=jax.ShapeDtypeStruct((NW, 16), jnp.float32),
    mesh=plsc.VectorSubcoreMesh(core_axis_name="c", subcore_axis_name="s"),
    compiler_params=pltpu.CompilerParams(needs_layout_passes=False),
    scratch_types=[
        pltpu.VMEM((T,), jnp.float32),
        pltpu.VMEM((T,), jnp.int32),
        pltpu.VMEM((4096,), jnp.int32),
        pltpu.VMEM((256,), jnp.int32),
        pltpu.VMEM((B,), jnp.int32),
        pltpu.VMEM((16,), jnp.float32),
        pltpu.SemaphoreType.DMA,
    ],
)
def _topk_mean_call(x_hbm, sl_hbm, out_hbm, *scratch):
    _topk_mean_kernel(x_hbm, sl_hbm, out_hbm, *scratch)


def kernel(x, seq_len):
    xf = jnp.reshape(x, (B, T)).astype(jnp.float32)
    slx = seq_len.astype(jnp.int32)
    out = _topk_mean_call(xf, slx)
    return jnp.reshape(out[:, :ROWS_PER_W], (B,))


# speedup vs baseline: 19.1137x; 4.8762x over previous
import functools

import jax
import jax.numpy as jnp
import numpy as np
from jax import lax
from jax.experimental import pallas as pl
from jax.experimental.pallas import tpu as pltpu
from jax.experimental.pallas import tpu_sc as plsc

B = 128
T = 32768
NW = 32
ROWS_PER_W = B // NW
NSTEP = T // 16
INT_MIN = np.int32(-2147483648)
M31 = np.int32(0x7FFFFFFF)


def _topk_mean_kernel(x_hbm, sl_hbm, out_hbm, vals_v, keys_v, hist_v,
                      tot_v, sl_v, res_v, sem):
    wid = lax.axis_index("s") * 2 + lax.axis_index("c")
    pltpu.sync_copy(sl_hbm, sl_v)

    lane = lax.iota(jnp.int32, 16)
    ones = jnp.ones((16,), jnp.int32)
    zf = jnp.zeros((16,), jnp.float32)
    zi = jnp.zeros((16,), jnp.int32)

    r0 = wid * ROWS_PER_W
    pltpu.async_copy(x_hbm.at[pl.ds(r0 * T, T)], vals_v, sem)

    def zero_hist():
        @plsc.parallel_loop(0, 256, unroll=8)
        def _(i):
            hist_v[pl.ds(i * 16, 16)] = zi

    def reduce_hist():
        @plsc.parallel_loop(0, 16, unroll=2)
        def _(c):
            bvec = (c * 16 + lane) * 16
            acc = zi
            for l in range(16):
                acc = acc + plsc.load_gather(hist_v, [bvec + l])
            tot_v[pl.ds(c * 16, 16)] = acc

    def scan_hist(kr):
        def chunk_body(i, carry):
            above, bstar, krn = carry
            c = 15 - i
            tvec = tot_v[pl.ds(c * 16, 16)]
            cs = plsc.cumsum(tvec)
            ctot = cs[15]
            s = above + ctot - cs + tvec
            mask = s >= kr
            nhit = plsc.all_reduce_population_count(mask)[0]
            hit = (bstar < 0) & (nhit > 0)
            lane_star = nhit - 1
            cs_star = jnp.sum(jnp.where(lane == lane_star, cs, 0))
            cum_above = above + ctot - cs_star
            bstar = jnp.where(hit, c * 16 + lane_star, bstar)
            krn = jnp.where(hit, kr - cum_above, krn)
            return above + ctot, bstar, krn
        _, bstar, krn = lax.fori_loop(
            0, 16, chunk_body, (np.int32(0), np.int32(-1), np.int32(1)))
        return bstar, krn

    def row_body(j, res):
        r = r0 + j
        pltpu.make_async_copy(x_hbm.at[pl.ds(r * T, T)], vals_v, sem).wait()

        chunk = sl_v[pl.ds((r // 16) * 16, 16)]
        sl = jnp.sum(jnp.where(lane == (r % 16), chunk, 0))
        k = sl // 16 + 1
        thr = (sl - lane + 15) >> 4

        zero_hist()

        @plsc.parallel_loop(0, NSTEP, unroll=8)
        def _(i):
            v = vals_v[pl.ds(i * 16, 16)]
            m = plsc.bitcast(v, jnp.int32)
            m = jnp.where(m >= 0, m, m ^ M31)
            m = jnp.where(i < thr, m, INT_MIN)
            keys_v[pl.ds(i * 16, 16)] = m
            idx = (lax.shift_right_logical(m ^ INT_MIN, 20) & 0xFF0) | lane
            plsc.addupdate_scatter(hist_v, [idx], ones)

        @pl.when(j < ROWS_PER_W - 1)
        def _():
            pltpu.async_copy(x_hbm.at[pl.ds((r + 1) * T, T)], vals_v, sem)

        reduce_hist()
        bstar, kr = scan_hist(k)
        prefix = bstar << 24

        for lvl in range(1, 4):
            shift = 24 - 8 * lvl
            zero_hist()
            phi = lax.shift_right_logical(prefix, shift + 8)

            @plsc.parallel_loop(0, NSTEP, unroll=8)
            def _(i):
                uk = keys_v[pl.ds(i * 16, 16)] ^ INT_MIN
                selm = lax.shift_right_logical(uk, shift + 8) == phi
                idx = ((lax.shift_right_logical(uk, shift) & 255) << 4) | lane
                plsc.addupdate_scatter(hist_v, [idx], ones, mask=selm)

            reduce_hist()
            bstar, kr = scan_hist(kr)
            prefix = prefix | (bstar << shift)

        tau_m = prefix ^ INT_MIN
        tau_b = jnp.where(tau_m >= 0, tau_m, tau_m ^ M31)
        tau_f = plsc.bitcast(jnp.broadcast_to(tau_b, (16,)), jnp.float32)

        @plsc.parallel_loop(0, NSTEP, step=8, carry=(zf,) * 8)
        def accs(i, accs):
            out = []
            for t in range(8):
                m = keys_v[pl.ds((i + t) * 16, 16)]
                bb = jnp.where(m >= 0, m, m ^ M31)
                v = plsc.bitcast(bb, jnp.float32)
                out.append(accs[t] + jnp.where(m > tau_m, v, 0.0))
            return tuple(out)

        acc = accs[0]
        for t in range(1, 8):
            acc = acc + accs[t]
        sum_gt = jnp.sum(acc)
        krf = jnp.broadcast_to(kr, (16,)).astype(jnp.float32)
        kf = jnp.broadcast_to(k, (16,)).astype(jnp.float32)
        rj = (sum_gt + krf * tau_f) / kf
        return jnp.where(lane == j, rj, res)

    res = lax.fori_loop(0, ROWS_PER_W, row_body, zf)
    res_v[...] = res
    pltpu.sync_copy(res_v, out_hbm.at[wid])


@functools.partial(
    pl.kernel,
    out_type=jax.ShapeDtypeStruct((NW, 16), jnp.float32),
    mesh=plsc.VectorSubcoreMesh(core_axis_name="c", subcore_axis_name="s"),
    compiler_params=pltpu.CompilerParams(needs_layout_passes=False),
    scratch_types=[
        pltpu.VMEM((T,), jnp.float32),
        pltpu.VMEM((T,), jnp.int32),
        pltpu.VMEM((4096,), jnp.int32),
        pltpu.VMEM((256,), jnp.int32),
        pltpu.VMEM((B,), jnp.int32),
        pltpu.VMEM((16,), jnp.float32),
        pltpu.SemaphoreType.DMA,
    ],
)
def _topk_mean_call(x_hbm, sl_hbm, out_hbm, *scratch):
    _topk_mean_kernel(x_hbm, sl_hbm, out_hbm, *scratch)


def kernel(x, seq_len):
    xf = jnp.reshape(x, (B * T,)).astype(jnp.float32)
    slx = seq_len.astype(jnp.int32)
    out = _topk_mean_call(xf, slx)
    return jnp.reshape(out[:, :ROWS_PER_W], (B,))
